# baseline (speedup 1.0000x reference)
import jax
import jax.numpy as jnp
from jax import lax
from jax.experimental import pallas as pl
from jax.experimental.pallas import tpu as pltpu
from jax.experimental.pallas import tpu_sc as plsc

VOCAB = 50257
H = 1280
S = 1024
B = 32

NUM_CORES = 2
NUM_SUBCORES = 16
NW = NUM_CORES * NUM_SUBCORES
SCHUNK = S // NW
LANES = 16
GROUPS = H // LANES

UNIT = 16
UPC = SCHUNK // UNIT
NU = B * UPC
NBUF = 4
T = NU // NBUF


def _body(ids_hbm, wte_hbm, wpe_hbm, out_hbm,
          idx_all, wpe_v, bufs, gsems, wsems):
  wid = lax.axis_index("s") * NUM_CORES + lax.axis_index("c")
  s0 = wid * SCHUNK

  def g_start(u, k):
    pltpu.async_copy(wte_hbm.at[idx_all.at[pl.ds(u * UNIT, UNIT)]],
                     bufs[k], gsems[k])

  def g_wait(u, k):
    pltpu.make_async_copy(wte_hbm.at[idx_all.at[pl.ds(u * UNIT, UNIT)]],
                          bufs[k], gsems[k]).wait()

  def out_slice(u, k):
    return out_hbm.at[pl.ds((u // UPC) * S + s0 + (k % UPC) * UNIT, UNIT)]

  def w_start(u, k):
    pltpu.async_copy(bufs[k], out_slice(u, k), wsems[k])

  def w_wait(u, k):
    pltpu.make_async_copy(bufs[k], out_slice(u, k), wsems[k]).wait()

  def add_unit(k):
    h = (k % UPC) * UNIT
    buf = bufs[k]

    @pl.loop(0, UNIT)
    def _(r):
      for j in range(GROUPS):
        plsc.addupdate(buf.at[r, pl.ds(j * LANES, LANES)],
                       wpe_v[h + r, pl.ds(j * LANES, LANES)])

  pltpu.sync_copy(ids_hbm.at[pl.ds(wid * B * SCHUNK, B * SCHUNK)], idx_all)
  pltpu.sync_copy(wpe_hbm.at[pl.ds(s0, SCHUNK)], wpe_v)

  g_start(0, 0)
  g_start(1, 1)

  @pl.loop(0, T)
  def _(t):
    u0 = NBUF * t
    for k in range(NBUF):
      u = u0 + k
      g_wait(u, k)
      add_unit(k)
      w_start(u, k)
      kn = (k + 2) % NBUF
      if k < 2:
        @pl.when(t > 0)
        def _():
          w_wait(u - 2, kn)
        g_start(u + 2, kn)
      else:
        @pl.when(t < T - 1)
        def _():
          w_wait(u - 2, kn)
          g_start(u + 2, kn)

  w_wait(NU - 4, 0)
  w_wait(NU - 3, 1)
  w_wait(NU - 2, 2)
  w_wait(NU - 1, 3)


@jax.jit
def kernel(input_ids, wte, wpe):
  ids = (input_ids.astype(jnp.int32)
         .reshape(B, NW, SCHUNK).swapaxes(0, 1).reshape(-1))
  mesh = plsc.VectorSubcoreMesh(core_axis_name="c", subcore_axis_name="s")
  run = pl.kernel(
      _body,
      out_type=jax.ShapeDtypeStruct((B * S, H), jnp.float32),
      mesh=mesh,
      scratch_types=[
          pltpu.VMEM((B * SCHUNK,), jnp.int32),
          pltpu.VMEM((SCHUNK, H), jnp.float32),
          [pltpu.VMEM((UNIT, H), jnp.float32) for _ in range(NBUF)],
          [pltpu.SemaphoreType.DMA for _ in range(NBUF)],
          [pltpu.SemaphoreType.DMA for _ in range(NBUF)],
      ],
  )
  out = run(ids, wte, wpe)
  return out.reshape(B, S, H)


# speedup vs baseline: 1.1338x; 1.1338x over previous
import jax
import jax.numpy as jnp
from jax import lax
from jax.experimental import pallas as pl
from jax.experimental.pallas import tpu as pltpu
from jax.experimental.pallas import tpu_sc as plsc

VOCAB = 50257
H = 1280
S = 1024
B = 32

NUM_CORES = 2
NUM_SUBCORES = 16
NW = NUM_CORES * NUM_SUBCORES
SCHUNK = S // NW
LANES = 16
GROUPS = H // LANES


def _body(ids_hbm, wte_hbm, wpe_hbm, out_hbm,
          idx_all, wpe_v, rows0, rows1, gsem0, gsem1, wsem0, wsem1):
  wid = lax.axis_index("s") * NUM_CORES + lax.axis_index("c")
  s0 = wid * SCHUNK

  def g_start(b, buf, sem):
    pltpu.async_copy(wte_hbm.at[idx_all.at[pl.ds(b * SCHUNK, SCHUNK)]],
                     buf, sem)

  def g_wait(b, buf, sem):
    pltpu.make_async_copy(wte_hbm.at[idx_all.at[pl.ds(b * SCHUNK, SCHUNK)]],
                          buf, sem).wait()

  def w_start(b, buf, sem):
    pltpu.async_copy(buf, out_hbm.at[pl.ds(b * S + s0, SCHUNK)], sem)

  def w_wait(b, buf, sem):
    pltpu.make_async_copy(
        buf, out_hbm.at[pl.ds(b * S + s0, SCHUNK)], sem).wait()

  def add_chunk(buf):
    @pl.loop(0, SCHUNK)
    def _(r):
      for j in range(GROUPS):
        plsc.addupdate(buf.at[r, pl.ds(j * LANES, LANES)],
                       wpe_v[r, pl.ds(j * LANES, LANES)])

  pltpu.sync_copy(ids_hbm.at[pl.ds(wid * B * SCHUNK, B * SCHUNK)], idx_all)
  pltpu.sync_copy(wpe_hbm.at[pl.ds(s0, SCHUNK)], wpe_v)

  g_start(0, rows0, gsem0)

  @pl.loop(0, B // 2)
  def _(t):
    b0 = 2 * t
    b1 = 2 * t + 1

    @pl.when(t > 0)
    def _():
      w_wait(b1, rows1, wsem1)

    g_start(b1, rows1, gsem1)
    g_wait(b0, rows0, gsem0)
    add_chunk(rows0)
    w_start(b0, rows0, wsem0)

    g_wait(b1, rows1, gsem1)
    add_chunk(rows1)

    @pl.when(t < B // 2 - 1)
    def _():
      w_wait(b0, rows0, wsem0)
      g_start(b0 + 2, rows0, gsem0)

    w_start(b1, rows1, wsem1)

  w_wait(B - 2, rows0, wsem0)
  w_wait(B - 1, rows1, wsem1)


@jax.jit
def kernel(input_ids, wte, wpe):
  ids = (input_ids.astype(jnp.int32)
         .reshape(B, NW, SCHUNK).swapaxes(0, 1).reshape(-1))
  mesh = plsc.VectorSubcoreMesh(core_axis_name="c", subcore_axis_name="s")
  run = pl.kernel(
      _body,
      out_type=jax.ShapeDtypeStruct((B * S, H), jnp.float32),
      mesh=mesh,
      scratch_types=[
          pltpu.VMEM((B * SCHUNK,), jnp.int32),
          pltpu.VMEM((SCHUNK, H), jnp.float32),
          pltpu.VMEM((SCHUNK, H), jnp.float32),
          pltpu.VMEM((SCHUNK, H), jnp.float32),
          pltpu.SemaphoreType.DMA,
          pltpu.SemaphoreType.DMA,
          pltpu.SemaphoreType.DMA,
          pltpu.SemaphoreType.DMA,
      ],
  )
  out = run(ids, wte, wpe)
  return out.reshape(B, S, H)


# speedup vs baseline: 1.1450x; 1.0099x over previous
import jax
import jax.numpy as jnp
from jax import lax
from jax.experimental import pallas as pl
from jax.experimental.pallas import tpu as pltpu
from jax.experimental.pallas import tpu_sc as plsc

VOCAB = 50257
H = 1280
S = 1024
B = 32

NUM_CORES = 2
NUM_SUBCORES = 16
NW = NUM_CORES * NUM_SUBCORES
SCHUNK = S // NW
LANES = 16
GROUPS = H // LANES


def _body(ids_hbm, wte_hbm, wpe_hbm, out_hbm,
          idx_all, wpe_v, rows0, rows1, gsem0, gsem1, wsem0, wsem1):
  wid = lax.axis_index("s") * NUM_CORES + lax.axis_index("c")
  s0 = wid * SCHUNK

  def g_start(b, buf, sem):
    pltpu.async_copy(wte_hbm.at[idx_all.at[pl.ds(b * SCHUNK, SCHUNK)]],
                     buf, sem)

  def g_wait(b, buf, sem):
    pltpu.make_async_copy(wte_hbm.at[idx_all.at[pl.ds(b * SCHUNK, SCHUNK)]],
                          buf, sem).wait()

  def w_start(b, buf, sem):
    pltpu.async_copy(buf, out_hbm.at[pl.ds(b * S + s0, SCHUNK)], sem)

  def w_wait(b, buf, sem):
    pltpu.make_async_copy(
        buf, out_hbm.at[pl.ds(b * S + s0, SCHUNK)], sem).wait()

  def add_chunk(buf):
    @pl.loop(0, SCHUNK)
    def _(r):
      for j in range(GROUPS):
        plsc.addupdate(buf.at[r, pl.ds(j * LANES, LANES)],
                       wpe_v[r, pl.ds(j * LANES, LANES)])

  pltpu.sync_copy(ids_hbm.at[pl.ds(wid * B * SCHUNK, B * SCHUNK)], idx_all)
  pltpu.sync_copy(wpe_hbm.at[pl.ds(s0, SCHUNK)], wpe_v)

  g_start(0, rows0, gsem0)

  @pl.loop(0, B // 2)
  def _(t):
    b0 = 2 * t
    b1 = 2 * t + 1

    @pl.when(t > 0)
    def _():
      w_wait(b1, rows1, wsem1)

    g_start(b1, rows1, gsem1)
    g_wait(b0, rows0, gsem0)
    add_chunk(rows0)
    w_start(b0, rows0, wsem0)

    g_wait(b1, rows1, gsem1)

    @pl.when(t < B // 2 - 1)
    def _():
      w_wait(b0, rows0, wsem0)
      g_start(b0 + 2, rows0, gsem0)

    add_chunk(rows1)
    w_start(b1, rows1, wsem1)

  w_wait(B - 2, rows0, wsem0)
  w_wait(B - 1, rows1, wsem1)


@jax.jit
def kernel(input_ids, wte, wpe):
  ids = (input_ids.astype(jnp.int32)
         .reshape(B, NW, SCHUNK).swapaxes(0, 1).reshape(-1))
  mesh = plsc.VectorSubcoreMesh(core_axis_name="c", subcore_axis_name="s")
  run = pl.kernel(
      _body,
      out_type=jax.ShapeDtypeStruct((B * S, H), jnp.float32),
      mesh=mesh,
      scratch_types=[
          pltpu.VMEM((B * SCHUNK,), jnp.int32),
          pltpu.VMEM((SCHUNK, H), jnp.float32),
          pltpu.VMEM((SCHUNK, H), jnp.float32),
          pltpu.VMEM((SCHUNK, H), jnp.float32),
          pltpu.SemaphoreType.DMA,
          pltpu.SemaphoreType.DMA,
          pltpu.SemaphoreType.DMA,
          pltpu.SemaphoreType.DMA,
      ],
  )
  out = run(ids, wte, wpe)
  return out.reshape(B, S, H)
